# baseline (speedup 1.0000x reference)
import functools

import jax
import jax.numpy as jnp
from jax import lax
from jax.experimental import pallas as pl
from jax.experimental.pallas import tpu as pltpu
from jax.experimental.pallas import tpu_sc as plsc

VOCAB = 1000000
EMBED = 64
B_TOTAL = 16384 * 20

_INFO = plsc.get_sparse_core_info()
_NC = _INFO.num_cores
_NS = _INFO.num_subcores
_NW = _NC * _NS
_PER_W = B_TOTAL // _NW
_CHUNK = 1024
_NCHUNK = _PER_W // _CHUNK


def _embed_kernel(idx_hbm, table_hbm, out_hbm, idx_v, rows_v, sem):
    wid = lax.axis_index("s") * _NC + lax.axis_index("c")
    base = wid * _PER_W
    for i in range(_NCHUNK):
        off = base + i * _CHUNK
        pltpu.sync_copy(idx_hbm.at[pl.ds(off, _CHUNK)], idx_v)
        pltpu.async_copy(table_hbm.at[idx_v], rows_v, sem).wait()
        pltpu.sync_copy(rows_v, out_hbm.at[pl.ds(off, _CHUNK)])


@functools.partial(jax.jit, static_argnames=())
def _embed(idx_flat, weight):
    mesh = plsc.VectorSubcoreMesh(core_axis_name="c", subcore_axis_name="s")
    k = functools.partial(
        pl.kernel,
        mesh=mesh,
        out_type=jax.ShapeDtypeStruct((B_TOTAL, EMBED), jnp.float32),
        scratch_types=[
            pltpu.VMEM((_CHUNK,), jnp.int32),
            pltpu.VMEM((_CHUNK, EMBED), jnp.float32),
            pltpu.SemaphoreType.DMA,
        ],
        compiler_params=pltpu.CompilerParams(use_tc_tiling_on_sc=False),
    )(_embed_kernel)
    return k(idx_flat, weight)


def kernel(input_ids, weight):
    idx_flat = input_ids.reshape(-1).astype(jnp.int32)
    out = _embed(idx_flat, weight)
    return out.reshape(input_ids.shape + (EMBED,))


# speedup vs baseline: 1.0053x; 1.0053x over previous
import functools

import jax
import jax.numpy as jnp
from jax import lax
from jax.experimental import pallas as pl
from jax.experimental.pallas import tpu as pltpu
from jax.experimental.pallas import tpu_sc as plsc

VOCAB = 1000000
EMBED = 64
B_TOTAL = 16384 * 20

_INFO = plsc.get_sparse_core_info()
_NC = _INFO.num_cores
_NS = _INFO.num_subcores
_NW = _NC * _NS
_PER_W = B_TOTAL // _NW
_CHUNK = 640
_NCHUNK = _PER_W // _CHUNK


def _embed_kernel(idx_hbm, table_hbm, out_hbm, idx_v, rows0, rows1, gsem0,
                  gsem1, osem0, osem1):
    wid = lax.axis_index("s") * _NC + lax.axis_index("c")
    base = wid * _PER_W
    pltpu.sync_copy(idx_hbm.at[pl.ds(base, _PER_W)], idx_v)
    rows = (rows0, rows1)
    gsem = (gsem0, gsem1)
    osem = (osem0, osem1)

    def gather(i):
        s = i % 2
        return pltpu.async_copy(
            table_hbm.at[idx_v.at[pl.ds(i * _CHUNK, _CHUNK)]], rows[s], gsem[s])

    def store(i):
        s = i % 2
        return pltpu.async_copy(
            rows[s], out_hbm.at[pl.ds(base + i * _CHUNK, _CHUNK)], osem[s])

    stores = [None, None]
    gather(0)
    for i in range(_NCHUNK):
        s = i % 2
        pltpu.make_async_copy(
            table_hbm.at[idx_v.at[pl.ds(i * _CHUNK, _CHUNK)]], rows[s],
            gsem[s]).wait()
        if i + 1 < _NCHUNK:
            if stores[(i + 1) % 2] is not None:
                stores[(i + 1) % 2].wait()
            gather(i + 1)
        stores[s] = store(i)
    stores[0].wait()
    stores[1].wait()


@functools.partial(jax.jit, static_argnames=())
def _embed(idx_flat, weight):
    mesh = plsc.VectorSubcoreMesh(core_axis_name="c", subcore_axis_name="s")
    k = functools.partial(
        pl.kernel,
        mesh=mesh,
        out_type=jax.ShapeDtypeStruct((B_TOTAL, EMBED), jnp.float32),
        scratch_types=[
            pltpu.VMEM((_PER_W,), jnp.int32),
            pltpu.VMEM((_CHUNK, EMBED), jnp.float32),
            pltpu.VMEM((_CHUNK, EMBED), jnp.float32),
            pltpu.SemaphoreType.DMA,
            pltpu.SemaphoreType.DMA,
            pltpu.SemaphoreType.DMA,
            pltpu.SemaphoreType.DMA,
        ],
        compiler_params=pltpu.CompilerParams(use_tc_tiling_on_sc=False),
    )(_embed_kernel)
    return k(idx_flat, weight)


def kernel(input_ids, weight):
    idx_flat = input_ids.reshape(-1).astype(jnp.int32)
    out = _embed(idx_flat, weight)
    return out.reshape(input_ids.shape + (EMBED,))
